# Initial kernel scaffold; baseline (speedup 1.0000x reference)
import functools

import jax
import jax.numpy as jnp
from jax import lax
from jax.experimental import pallas as pl
from jax.experimental.pallas import tpu as pltpu
from jax.experimental.pallas import tpu_sc as plsc

N = 10000
E = 160000
H = 256
HH = 128
TD = 128
L = 4

NC = 2
NS = 16
CH = 128

E2 = 163840
PAD = E2 - E
EPT = E2 // NS
NCHUNK = EPT // CH
NP = 10016
RPS = NP // NS
RPSO = N // NS

_mesh = plsc.VectorSubcoreMesh(core_axis_name="c", subcore_axis_name="s")


def _fill(ref, rows, cols, val):
    def body(j, _):
        r = j // (cols // 16)
        k = j % (cols // 16)
        ref[r, pl.ds(k * 16, 16)] = jnp.full((16,), val, jnp.float32)
        return 0
    lax.fori_loop(0, rows * (cols // 16), body, 0)


def _zero_acc_slice(zbuf, acc, base, total):
    off = 0
    while off < total:
        n = min(128, total - off)
        pltpu.sync_copy(zbuf.at[pl.ds(0, n)], acc.at[pl.ds(base + off, n)])
        off += n


@functools.partial(
    pl.kernel,
    out_type=jax.ShapeDtypeStruct((NC * NP, 16), jnp.float32),
    mesh=_mesh,
    scratch_types=[
        pltpu.VMEM_SHARED((NP, 16), jnp.float32),
        pltpu.VMEM((CH, 16), jnp.float32),
        pltpu.VMEM((CH,), jnp.int32),
    ],
)
def _sc_degrees(idx_hbm, out_hbm, acc, ones, idxb):
    c = lax.axis_index("c")
    s = lax.axis_index("s")
    _fill(ones, CH, 16, 0.0)
    _zero_acc_slice(ones, acc, s * RPS, RPS)
    _fill(ones, CH, 16, 1.0)
    plsc.subcore_barrier()
    base = c * E2 + s * EPT

    def body(k, _):
        pltpu.sync_copy(idx_hbm.at[pl.ds(base + k * CH, CH)], idxb)
        pltpu.sync_copy(ones, acc.at[idxb], add=True)
        return 0

    lax.fori_loop(0, NCHUNK, body, 0)
    plsc.subcore_barrier()
    pltpu.sync_copy(acc.at[pl.ds(s * RPS, RPS)],
                    out_hbm.at[pl.ds(c * NP + s * RPS, RPS)])


@functools.partial(
    pl.kernel,
    out_type=jax.ShapeDtypeStruct((NC * N, HH), jnp.float32),
    mesh=_mesh,
    scratch_types=[
        pltpu.VMEM_SHARED((NP, HH), jnp.float32),
        pltpu.VMEM((CH, HH), jnp.float32),
        pltpu.VMEM((CH, HH), jnp.float32),
        pltpu.VMEM((CH,), jnp.int32),
        pltpu.VMEM((CH,), jnp.int32),
        pltpu.VMEM((CH,), jnp.int32),
        pltpu.VMEM((CH,), jnp.int32),
        pltpu.SemaphoreType.DMA,
        pltpu.SemaphoreType.DMA,
    ],
)
def _sc_agg(hs_hbm, src_hbm, dst_hbm, out_hbm,
            acc, rows0, rows1, sidx0, sidx1, didx0, didx1, sem0, sem1):
    c = lax.axis_index("c")
    s = lax.axis_index("s")
    _fill(rows0, CH, HH, 0.0)
    _zero_acc_slice(rows0, acc, s * RPS, RPS)
    plsc.subcore_barrier()

    ebase = c * E2 + s * EPT
    dbase = s * EPT

    def issue(k, sidx, didx, rows, sem):
        pltpu.sync_copy(src_hbm.at[pl.ds(ebase + k * CH, CH)], sidx)
        pltpu.sync_copy(dst_hbm.at[pl.ds(dbase + k * CH, CH)], didx)
        pltpu.async_copy(hs_hbm.at[sidx], rows, sem)

    def drain_scatter(rows, didx, sem):
        pltpu.make_async_copy(hs_hbm.at[pl.ds(0, CH)], rows, sem).wait()
        pltpu.sync_copy(rows, acc.at[didx], add=True)

    issue(0, sidx0, didx0, rows0, sem0)
    issue(1, sidx1, didx1, rows1, sem1)

    def body(g, _):
        drain_scatter(rows0, didx0, sem0)
        issue(2 * g + 2, sidx0, didx0, rows0, sem0)
        drain_scatter(rows1, didx1, sem1)
        issue(2 * g + 3, sidx1, didx1, rows1, sem1)
        return 0

    lax.fori_loop(0, NCHUNK // 2 - 1, body, 0)
    drain_scatter(rows0, didx0, sem0)
    drain_scatter(rows1, didx1, sem1)
    plsc.subcore_barrier()
    pltpu.sync_copy(acc.at[pl.ds(s * RPSO, RPSO)],
                    out_hbm.at[pl.ds(c * N + s * RPSO, RPSO)])


def _gelu(x):
    return x * 0.5 * (1.0 + lax.erf(x * (2.0 ** -0.5)))


def _tc_embed_body(xin, tf, freqs, degs, degd, tW1, tb1, tW2, tb2, inW, inb,
                   h0, hs, ns_o, nd_o):
    ang = tf[0, 0] * freqs[...]
    emb = jnp.concatenate([jnp.sin(ang), jnp.cos(ang)], axis=1)
    th = jnp.dot(emb, tW1[...], preferred_element_type=jnp.float32) + tb1[...]
    th = th * jax.nn.sigmoid(th)
    temb = jnp.dot(th, tW2[...], preferred_element_type=jnp.float32) + tb2[...]
    ns = lax.rsqrt(jnp.maximum(degs[...], 1.0))
    nd = lax.rsqrt(jnp.maximum(degd[...], 1.0))
    h = jnp.dot(xin[...], inW[...], preferred_element_type=jnp.float32)
    h = _gelu(h + inb[...] + temb)
    h0[...] = h
    hsv = h * ns
    hs[0] = hsv[:, :HH]
    hs[1] = hsv[:, HH:]
    ns_o[...] = ns
    nd_o[...] = nd


def _tc_layer_body(first, last, agg, nd, ns, h_in, W, b, g, beta, hW, hb,
                   *outs):
    aggc = jnp.concatenate([agg[0], agg[1]], axis=1) * nd[...]
    z = jnp.dot(aggc, W[...], preferred_element_type=jnp.float32) + b[...]
    mu = jnp.mean(z, axis=-1, keepdims=True)
    var = jnp.mean((z - mu) ** 2, axis=-1, keepdims=True)
    h = _gelu((z - mu) * lax.rsqrt(var + 1e-5) * g[...] + beta[...])
    if not first:
        h = h + h_in[...]
    if last:
        outs[0][...] = (jnp.dot(h, hW[...], preferred_element_type=jnp.float32)
                        + hb[...])
    else:
        outs[0][...] = h
        hsv = h * ns[...]
        outs[1][0] = hsv[:, :HH]
        outs[1][1] = hsv[:, HH:]


def kernel(x_t, t, cond, edge_index, tW1, tb1, tW2, tb2, inW, inb, convW,
           convb, lnG, lnB, headW, headb):
    src = edge_index[0].astype(jnp.int32)
    dst = edge_index[1].astype(jnp.int32)
    padN = jnp.full((PAD,), N, jnp.int32)
    idxdeg = jnp.concatenate([src, padN, dst, padN])
    srcg = jnp.concatenate([src, jnp.zeros((PAD,), jnp.int32)])
    srcg2 = jnp.concatenate([srcg, srcg + N])
    dstp = jnp.concatenate([dst, padN])

    degs_all = _sc_degrees(idxdeg)
    degv = degs_all.reshape(NC, NP, 16)
    deg_s = degv[0, :N, 0:1]
    deg_d = degv[1, :N, 0:1]

    R = 1000
    G = N // R
    xin = jnp.concatenate([x_t, cond], axis=1)
    tf = t.astype(jnp.float32).reshape(1, 1)
    half = TD // 2
    freqs = jnp.exp(jnp.arange(half, dtype=jnp.float32)
                    * (-jnp.log(10000.0) / half)).reshape(1, half)

    full2 = lambda a: pl.BlockSpec(a, lambda i: (0, 0))
    rows2 = lambda d: pl.BlockSpec((R, d), lambda i: (i, 0))
    rows3 = lambda d: pl.BlockSpec((NC, R, d), lambda i: (0, i, 0))

    h0, hs, ns, nd = pl.pallas_call(
        _tc_embed_body,
        grid=(G,),
        in_specs=[
            rows2(4), full2((1, 1)), full2((1, half)),
            rows2(1), rows2(1),
            full2((TD, H)), full2((1, H)), full2((H, H)), full2((1, H)),
            full2((4, H)), full2((1, H)),
        ],
        out_specs=[rows2(H), rows3(HH), rows2(1), rows2(1)],
        out_shape=[
            jax.ShapeDtypeStruct((N, H), jnp.float32),
            jax.ShapeDtypeStruct((NC, N, HH), jnp.float32),
            jax.ShapeDtypeStruct((N, 1), jnp.float32),
            jax.ShapeDtypeStruct((N, 1), jnp.float32),
        ],
    )(xin, tf, freqs, deg_s, deg_d, tW1, tb1.reshape(1, H), tW2,
      tb2.reshape(1, H), inW, inb.reshape(1, H))

    headWp = jnp.pad(headW, ((0, 0), (0, HH - 2)))
    headbp = jnp.pad(headb, (0, HH - 2)).reshape(1, HH)

    h = h0
    eps_pad = None
    for i in range(L):
        aggf = _sc_agg(hs.reshape(NC * N, HH), srcg2, dstp)
        agg = aggf.reshape(NC, N, HH)
        first, last = (i == 0), (i == L - 1)
        if last:
            out_specs = [rows2(HH)]
            out_shape = [jax.ShapeDtypeStruct((N, HH), jnp.float32)]
        else:
            out_specs = [rows2(H), rows3(HH)]
            out_shape = [
                jax.ShapeDtypeStruct((N, H), jnp.float32),
                jax.ShapeDtypeStruct((NC, N, HH), jnp.float32),
            ]
        res = pl.pallas_call(
            functools.partial(_tc_layer_body, first, last),
            grid=(G,),
            in_specs=[
                rows3(HH), rows2(1), rows2(1), rows2(H),
                full2((H, H)), full2((1, H)), full2((1, H)), full2((1, H)),
                full2((H, HH)), full2((1, HH)),
            ],
            out_specs=out_specs,
            out_shape=out_shape,
        )(agg, nd, ns, h, convW[i], convb[i].reshape(1, H),
          lnG[i].reshape(1, H), lnB[i].reshape(1, H), headWp, headbp)
        if last:
            eps_pad = res[0]
        else:
            h, hs = res
    return eps_pad[:, :2]


# speedup vs baseline: 2.3626x; 2.3626x over previous
import functools

import jax
import jax.numpy as jnp
from jax import lax
from jax.experimental import pallas as pl
from jax.experimental.pallas import tpu as pltpu
from jax.experimental.pallas import tpu_sc as plsc

N = 10000
E = 160000
H = 256
HH = 128
TD = 128
L = 4

NC = 2
NS = 16
CH = 128

E2 = 163840
PAD = E2 - E
EPT = E2 // NS
NCHUNK = EPT // CH
NP = 10112
RPS = NP // NS
RPSO = 520

_mesh = plsc.VectorSubcoreMesh(core_axis_name="c", subcore_axis_name="s")


def _fill(ref, rows, cols, val):
    def body(j, _):
        r = j // (cols // 16)
        k = j % (cols // 16)
        ref[r, pl.ds(k * 16, 16)] = jnp.full((16,), val, jnp.float32)
        return 0
    lax.fori_loop(0, rows * (cols // 16), body, 0)


def _zero_acc_slice(zbuf, acc, base, total):
    off = 0
    while off < total:
        n = min(128, total - off)
        pltpu.sync_copy(zbuf.at[pl.ds(0, n)], acc.at[pl.ds(base + off, n)])
        off += n


@functools.partial(
    pl.kernel,
    out_type=jax.ShapeDtypeStruct((NC * NP, 16), jnp.float32),
    mesh=_mesh,
    scratch_types=[
        pltpu.VMEM_SHARED((NP, 16), jnp.float32),
        pltpu.VMEM((CH, 16), jnp.float32),
        pltpu.VMEM((CH,), jnp.int32),
    ],
)
def _sc_degrees(idx_hbm, out_hbm, acc, ones, idxb):
    c = lax.axis_index("c")
    s = lax.axis_index("s")
    _fill(ones, CH, 16, 0.0)
    _zero_acc_slice(ones, acc, s * RPS, RPS)
    _fill(ones, CH, 16, 1.0)
    plsc.subcore_barrier()
    base = c * E2 + s * EPT

    def body(k, _):
        pltpu.sync_copy(idx_hbm.at[pl.ds(base + k * CH, CH)], idxb)
        pltpu.sync_copy(ones, acc.at[idxb], add=True)
        return 0

    lax.fori_loop(0, NCHUNK, body, 0)
    plsc.subcore_barrier()
    pltpu.sync_copy(acc.at[pl.ds(s * RPS, RPS)],
                    out_hbm.at[pl.ds(c * NP + s * RPS, RPS)])


@functools.partial(
    pl.kernel,
    out_type=jax.ShapeDtypeStruct((NC * N, HH), jnp.float32),
    mesh=_mesh,
    scratch_types=[
        pltpu.VMEM_SHARED((NP, HH), jnp.float32),
        pltpu.VMEM((CH, HH), jnp.float32),
        pltpu.VMEM((CH, HH), jnp.float32),
        pltpu.VMEM((CH,), jnp.int32),
        pltpu.VMEM((CH,), jnp.int32),
        pltpu.VMEM((CH,), jnp.int32),
        pltpu.VMEM((CH,), jnp.int32),
        pltpu.SemaphoreType.DMA,
        pltpu.SemaphoreType.DMA,
    ],
)
def _sc_agg(hs_hbm, src_hbm, dst_hbm, out_hbm,
            acc, rows0, rows1, sidx0, sidx1, didx0, didx1, sem0, sem1):
    c = lax.axis_index("c")
    s = lax.axis_index("s")
    _fill(rows0, CH, HH, 0.0)
    _zero_acc_slice(rows0, acc, s * RPS, RPS)
    plsc.subcore_barrier()

    ebase = c * E2 + s * EPT
    dbase = s * EPT

    def issue(k, sidx, didx, rows, sem):
        pltpu.sync_copy(src_hbm.at[pl.ds(ebase + k * CH, CH)], sidx)
        pltpu.sync_copy(dst_hbm.at[pl.ds(dbase + k * CH, CH)], didx)
        pltpu.async_copy(hs_hbm.at[sidx], rows, sem).wait()

    def drain_scatter(rows, didx, sem):
        pltpu.sync_copy(rows, acc.at[didx], add=True)

    issue(0, sidx0, didx0, rows0, sem0)
    issue(1, sidx1, didx1, rows1, sem1)

    def body(g, _):
        drain_scatter(rows0, didx0, sem0)
        issue(2 * g + 2, sidx0, didx0, rows0, sem0)
        drain_scatter(rows1, didx1, sem1)
        issue(2 * g + 3, sidx1, didx1, rows1, sem1)
        return 0

    lax.fori_loop(0, NCHUNK // 2 - 1, body, 0)
    drain_scatter(rows0, didx0, sem0)
    drain_scatter(rows1, didx1, sem1)
    plsc.subcore_barrier()

    @pl.when(s < NS - 1)
    def _():
        pltpu.sync_copy(acc.at[pl.ds(s * RPS, RPS)],
                        out_hbm.at[pl.ds(c * N + s * RPS, RPS)])

    @pl.when(s == NS - 1)
    def _():
        pltpu.sync_copy(acc.at[pl.ds((NS - 1) * RPS, RPSO)],
                        out_hbm.at[pl.ds(c * N + (NS - 1) * RPS, RPSO)])


def _gelu(x):
    return x * 0.5 * (1.0 + lax.erf(x * (2.0 ** -0.5)))


def _tc_embed_body(xin, tf, freqs, degs, degd, tW1, tb1, tW2, tb2, inW, inb,
                   h0, hs, ns_o, nd_o):
    ang = tf[0, 0] * freqs[...]
    emb = jnp.concatenate([jnp.sin(ang), jnp.cos(ang)], axis=1)
    th = jnp.dot(emb, tW1[...], preferred_element_type=jnp.float32) + tb1[...]
    th = th * jax.nn.sigmoid(th)
    temb = jnp.dot(th, tW2[...], preferred_element_type=jnp.float32) + tb2[...]
    ns = lax.rsqrt(jnp.maximum(degs[...], 1.0))
    nd = lax.rsqrt(jnp.maximum(degd[...], 1.0))
    h = jnp.dot(xin[...], inW[...], preferred_element_type=jnp.float32)
    h = _gelu(h + inb[...] + temb)
    h0[...] = h
    hsv = h * ns
    hs[0] = hsv[:, :HH]
    hs[1] = hsv[:, HH:]
    ns_o[...] = ns
    nd_o[...] = nd


def _tc_layer_body(first, last, agg, nd, ns, h_in, W, b, g, beta, hW, hb,
                   *outs):
    aggc = jnp.concatenate([agg[0], agg[1]], axis=1) * nd[...]
    z = jnp.dot(aggc, W[...], preferred_element_type=jnp.float32) + b[...]
    mu = jnp.mean(z, axis=-1, keepdims=True)
    var = jnp.mean((z - mu) ** 2, axis=-1, keepdims=True)
    h = _gelu((z - mu) * lax.rsqrt(var + 1e-5) * g[...] + beta[...])
    if not first:
        h = h + h_in[...]
    if last:
        outs[0][...] = (jnp.dot(h, hW[...], preferred_element_type=jnp.float32)
                        + hb[...])
    else:
        outs[0][...] = h
        hsv = h * ns[...]
        outs[1][0] = hsv[:, :HH]
        outs[1][1] = hsv[:, HH:]


def kernel(x_t, t, cond, edge_index, tW1, tb1, tW2, tb2, inW, inb, convW,
           convb, lnG, lnB, headW, headb):
    src = edge_index[0].astype(jnp.int32)
    dst = edge_index[1].astype(jnp.int32)
    padN = jnp.full((PAD,), N, jnp.int32)
    idxdeg = jnp.concatenate([src, padN, dst, padN])
    srcg = jnp.concatenate([src, jnp.zeros((PAD,), jnp.int32)])
    srcg2 = jnp.concatenate([srcg, srcg + N])
    dstp = jnp.concatenate([dst, padN])

    degs_all = _sc_degrees(idxdeg)
    degv = degs_all.reshape(NC, NP, 16)
    deg_s = degv[0, :N, 0:1]
    deg_d = degv[1, :N, 0:1]

    R = 1000
    G = N // R
    xin = jnp.concatenate([x_t, cond], axis=1)
    tf = t.astype(jnp.float32).reshape(1, 1)
    half = TD // 2
    freqs = jnp.exp(jnp.arange(half, dtype=jnp.float32)
                    * (-jnp.log(10000.0) / half)).reshape(1, half)

    full2 = lambda a: pl.BlockSpec(a, lambda i: (0, 0))
    rows2 = lambda d: pl.BlockSpec((R, d), lambda i: (i, 0))
    rows3 = lambda d: pl.BlockSpec((NC, R, d), lambda i: (0, i, 0))

    h0, hs, ns, nd = pl.pallas_call(
        _tc_embed_body,
        grid=(G,),
        in_specs=[
            rows2(4), full2((1, 1)), full2((1, half)),
            rows2(1), rows2(1),
            full2((TD, H)), full2((1, H)), full2((H, H)), full2((1, H)),
            full2((4, H)), full2((1, H)),
        ],
        out_specs=[rows2(H), rows3(HH), rows2(1), rows2(1)],
        out_shape=[
            jax.ShapeDtypeStruct((N, H), jnp.float32),
            jax.ShapeDtypeStruct((NC, N, HH), jnp.float32),
            jax.ShapeDtypeStruct((N, 1), jnp.float32),
            jax.ShapeDtypeStruct((N, 1), jnp.float32),
        ],
    )(xin, tf, freqs, deg_s, deg_d, tW1, tb1.reshape(1, H), tW2,
      tb2.reshape(1, H), inW, inb.reshape(1, H))

    headWp = jnp.pad(headW, ((0, 0), (0, HH - 2)))
    headbp = jnp.pad(headb, (0, HH - 2)).reshape(1, HH)

    h = h0
    eps_pad = None
    for i in range(L):
        aggf = _sc_agg(hs.reshape(NC * N, HH), srcg2, dstp)
        agg = aggf.reshape(NC, N, HH)
        first, last = (i == 0), (i == L - 1)
        if last:
            out_specs = [rows2(HH)]
            out_shape = [jax.ShapeDtypeStruct((N, HH), jnp.float32)]
        else:
            out_specs = [rows2(H), rows3(HH)]
            out_shape = [
                jax.ShapeDtypeStruct((N, H), jnp.float32),
                jax.ShapeDtypeStruct((NC, N, HH), jnp.float32),
            ]
        res = pl.pallas_call(
            functools.partial(_tc_layer_body, first, last),
            grid=(G,),
            in_specs=[
                rows3(HH), rows2(1), rows2(1), rows2(H),
                full2((H, H)), full2((1, H)), full2((1, H)), full2((1, H)),
                full2((H, HH)), full2((1, HH)),
            ],
            out_specs=out_specs,
            out_shape=out_shape,
        )(agg, nd, ns, h, convW[i], convb[i].reshape(1, H),
          lnG[i].reshape(1, H), lnB[i].reshape(1, H), headWp, headbp)
        if last:
            eps_pad = res[0]
        else:
            h, hs = res
    return eps_pad[:, :2]
